# baseline (speedup 1.0000x reference)
import functools

import jax
import jax.numpy as jnp
from jax import lax
from jax.experimental import pallas as pl
from jax.experimental.pallas import tpu as pltpu
from jax.experimental.pallas import tpu_sc as plsc

DIM = 768
E = 8
B = 1
S = 2048
T = B * S
H = 4 * DIM

BLK = 512
NPAD = 4864
HCH = 2
HB = H // HCH

NW = 32
TPW = T // NW
NEG = -1e30


def _router_body(x_ref, wr_ref, br_ref, wn_ref, bn_ref, eps_ref,
                 i1_ref, i2_ref, g1_ref, g2_ref):
    x = x_ref[...]
    logits = lax.dot_general(
        x, wr_ref[...], (((1,), (1,)), ((), ())),
        preferred_element_type=jnp.float32) + br_ref[...]
    noise = lax.dot_general(
        x, wn_ref[...], (((1,), (1,)), ((), ())),
        preferred_element_type=jnp.float32) + bn_ref[...]
    sp = jnp.log1p(jnp.exp(-jnp.abs(noise))) + jnp.maximum(noise, 0.0)
    noisy = logits + eps_ref[...] * sp

    iota = lax.broadcasted_iota(jnp.int32, (T, E), 1)
    m1 = jnp.max(noisy, axis=1, keepdims=True)
    i1 = jnp.min(jnp.where(noisy == m1, iota, E), axis=1, keepdims=True)
    masked = jnp.where(iota == i1, NEG, noisy)
    m2 = jnp.max(masked, axis=1, keepdims=True)
    i2 = jnp.min(jnp.where(masked == m2, iota, E), axis=1, keepdims=True)
    t = jnp.exp(m2 - m1)
    g1 = 1.0 / (1.0 + t)
    g2 = t / (1.0 + t)
    i1_ref[...] = i1
    i2_ref[...] = i2
    g1_ref[...] = g1
    g2_ref[...] = g2


def _router(xf, Wr, br, Wn, bn, eps):
    return pl.pallas_call(
        _router_body,
        out_shape=[
            jax.ShapeDtypeStruct((T, 1), jnp.int32),
            jax.ShapeDtypeStruct((T, 1), jnp.int32),
            jax.ShapeDtypeStruct((T, 1), jnp.float32),
            jax.ShapeDtypeStruct((T, 1), jnp.float32),
        ],
        in_specs=[
            pl.BlockSpec((T, DIM), lambda: (0, 0)),
            pl.BlockSpec((E, DIM), lambda: (0, 0)),
            pl.BlockSpec((1, E), lambda: (0, 0)),
            pl.BlockSpec((E, DIM), lambda: (0, 0)),
            pl.BlockSpec((1, E), lambda: (0, 0)),
            pl.BlockSpec((T, E), lambda: (0, 0)),
        ],
        out_specs=[
            pl.BlockSpec((T, 1), lambda: (0, 0)),
            pl.BlockSpec((T, 1), lambda: (0, 0)),
            pl.BlockSpec((T, 1), lambda: (0, 0)),
            pl.BlockSpec((T, 1), lambda: (0, 0)),
        ],
    )(xf, Wr, br.reshape(1, E), Wn, bn.reshape(1, E), eps)


def _dispatch_body(i1_hbm, i2_hbm, x_hbm,
                   pos1_hbm, pos2_hbm, xs_hbm, rs_hbm, nt_hbm,
                   ia_v, ib_v, mya_v, myb_v, pos1_v, pos2_v, xrows_v,
                   d0_v, d1_v):
    cid = lax.axis_index("c")
    sid = lax.axis_index("s")
    wid = cid * 16 + sid
    base_t = wid * TPW
    lane = lax.broadcasted_iota(jnp.int32, (16,), 0)
    zero16 = jnp.zeros((16,), jnp.int32)
    one16 = jnp.full((16,), 1, jnp.int32)

    def _splat(s):
        return lax.broadcast_in_dim(s, (16,), ())

    pltpu.sync_copy(i1_hbm, ia_v)
    pltpu.sync_copy(i2_hbm, ib_v)
    pltpu.sync_copy(i1_hbm.at[pl.ds(base_t, TPW)], mya_v)
    pltpu.sync_copy(i2_hbm.at[pl.ds(base_t, TPW)], myb_v)
    pltpu.sync_copy(x_hbm.at[pl.ds(base_t, TPW)], xrows_v)

    def hist_body(k, carry):
        va = ia_v[pl.ds(k * 16, 16)]
        vb = ib_v[pl.ds(k * 16, 16)]
        selv = _splat((k < wid * (TPW // 16)).astype(jnp.int32))
        out = []
        for e in range(E):
            inc = (va == e).astype(jnp.int32) + (vb == e).astype(jnp.int32)
            out.append(carry[e] + inc)
            out.append(carry[E + e] + inc * selv)
        return tuple(out[0::2]) + tuple(out[1::2])

    acc = lax.fori_loop(0, T // 16, hist_body, (zero16,) * (2 * E))
    tot = zero16
    pre = zero16
    for e in range(E):
        tot = tot + jnp.where(lane == e, _splat(jnp.sum(acc[e])), zero16)
        pre = pre + jnp.where(lane == e, _splat(jnp.sum(acc[E + e])),
                              zero16)

    nb8 = lax.shift_right_logical(tot + _splat(jnp.int32(7)),
                                  jnp.full((16,), 3, jnp.int32))
    cum8 = plsc.cumsum(nb8)
    seg_base = (cum8 - nb8) * _splat(jnp.int32(8))
    start = seg_base + pre

    @pl.when(wid == 0)
    def _():
        ntile = lax.shift_right_logical(tot + _splat(jnp.int32(BLK - 1)),
                                        jnp.full((16,), 9, jnp.int32))
        d0_v[pl.ds(0, 16)] = seg_base
        d1_v[pl.ds(0, 16)] = ntile
        pltpu.sync_copy(d0_v, rs_hbm)
        pltpu.sync_copy(d1_v, nt_hbm)

    cur = start
    for slot in range(2):
        src = mya_v if slot == 0 else myb_v
        dst = pos1_v if slot == 0 else pos2_v
        for v in range(TPW // 16):
            ev = src[pl.ds(v * 16, 16)]
            pos = zero16
            for e in range(E):
                m = ev == e
                ind = m.astype(jnp.int32)
                r = jnp.cumsum(ind)
                ce = jnp.sum(jnp.where(lane == e, cur, zero16))
                pos = jnp.where(m, _splat(ce) + r - one16, pos)
                cur = cur + jnp.where(lane == e, _splat(jnp.sum(ind)),
                                      zero16)
            dst[pl.ds(v * 16, 16)] = pos

    pltpu.sync_copy(pos1_v, pos1_hbm.at[pl.ds(base_t, TPW)])
    pltpu.sync_copy(pos2_v, pos2_hbm.at[pl.ds(base_t, TPW)])
    pltpu.sync_copy(xrows_v, xs_hbm.at[pos1_v])
    pltpu.sync_copy(xrows_v, xs_hbm.at[pos2_v])


def _dispatch(i1f, i2f, xf):
    mesh = plsc.VectorSubcoreMesh(core_axis_name="c", subcore_axis_name="s")
    fn = functools.partial(
        pl.kernel,
        out_type=[
            jax.ShapeDtypeStruct((T,), jnp.int32),
            jax.ShapeDtypeStruct((T,), jnp.int32),
            jax.ShapeDtypeStruct((NPAD, DIM), jnp.float32),
            jax.ShapeDtypeStruct((16,), jnp.int32),
            jax.ShapeDtypeStruct((16,), jnp.int32),
        ],
        mesh=mesh,
        scratch_types=[
            pltpu.VMEM((T,), jnp.int32),
            pltpu.VMEM((T,), jnp.int32),
            pltpu.VMEM((TPW,), jnp.int32),
            pltpu.VMEM((TPW,), jnp.int32),
            pltpu.VMEM((TPW,), jnp.int32),
            pltpu.VMEM((TPW,), jnp.int32),
            pltpu.VMEM((TPW, DIM), jnp.float32),
            pltpu.VMEM((16,), jnp.int32),
            pltpu.VMEM((16,), jnp.int32),
        ],
        compiler_params=pltpu.CompilerParams(needs_layout_passes=False),
    )(_dispatch_body)
    return fn(i1f, i2f, xf)


def _ffn_body(rs_ref, nt_ref,
              xs_ref, w1_ref, b1_ref, w2_ref, b2_ref, y_ref):
    e = pl.program_id(0)
    c = pl.program_id(1)
    rs = rs_ref[e]
    nt = nt_ref[e]

    def tile(t, carry):
        s = pl.multiple_of(rs + t * BLK, 8)
        xb = xs_ref[pl.ds(s, BLK), :]
        h = lax.dot_general(
            xb, w1_ref[0], (((1,), (1,)), ((), ())),
            preferred_element_type=jnp.float32) + b1_ref[0]
        h = jnp.maximum(h, 0.0)
        o = lax.dot_general(
            h, w2_ref[0], (((1,), (1,)), ((), ())),
            preferred_element_type=jnp.float32)

        @pl.when(c == 0)
        def _():
            y_ref[pl.ds(s, BLK), :] = o + b2_ref[0]

        @pl.when(c != 0)
        def _():
            y_ref[pl.ds(s, BLK), :] += o

        return carry

    lax.fori_loop(0, nt, tile, 0)


def _ffn(rs, nt, xs, W1, b1, W2, b2):
    grid_spec = pltpu.PrefetchScalarGridSpec(
        num_scalar_prefetch=2,
        grid=(E, HCH),
        in_specs=[
            pl.BlockSpec((NPAD, DIM), lambda e, c, rs, nt: (0, 0)),
            pl.BlockSpec((1, HB, DIM), lambda e, c, rs, nt: (e, c, 0)),
            pl.BlockSpec((1, 1, HB), lambda e, c, rs, nt: (e, 0, c)),
            pl.BlockSpec((1, DIM, HB), lambda e, c, rs, nt: (e, 0, c)),
            pl.BlockSpec((1, 1, DIM), lambda e, c, rs, nt: (e, 0, 0)),
        ],
        out_specs=pl.BlockSpec((NPAD, DIM), lambda e, c, rs, nt: (0, 0)),
    )
    return pl.pallas_call(
        _ffn_body,
        grid_spec=grid_spec,
        out_shape=jax.ShapeDtypeStruct((NPAD, DIM), jnp.float32),
        compiler_params=pltpu.CompilerParams(
            dimension_semantics=("arbitrary", "arbitrary"),
            vmem_limit_bytes=56 * 1024 * 1024,
        ),
    )(rs, nt, xs, W1, b1.reshape(E, 1, H), W2, b2.reshape(E, 1, DIM))


def _combine_body(y_hbm, pos1_hbm, pos2_hbm, g1_hbm, g2_hbm, out_hbm,
                  idx1_v, idx2_v, g1_v, g2_v, rows1_v, rows2_v):
    cid = lax.axis_index("c")
    sid = lax.axis_index("s")
    wid = cid * 16 + sid
    base_t = wid * TPW

    pltpu.sync_copy(pos1_hbm.at[pl.ds(base_t, TPW)], idx1_v)
    pltpu.sync_copy(pos2_hbm.at[pl.ds(base_t, TPW)], idx2_v)
    pltpu.sync_copy(g1_hbm.at[pl.ds(base_t, TPW)], g1_v)
    pltpu.sync_copy(g2_hbm.at[pl.ds(base_t, TPW)], g2_v)
    pltpu.sync_copy(y_hbm.at[idx1_v], rows1_v)
    pltpu.sync_copy(y_hbm.at[idx2_v], rows2_v)

    def cbody(i, carry):
        isplat = jnp.broadcast_to(i, (16,))
        ga = plsc.load_gather(g1_v, [isplat])
        gb = plsc.load_gather(g2_v, [isplat])
        for c in range(DIM // 16):
            r1 = rows1_v[i, pl.ds(c * 16, 16)]
            r2 = rows2_v[i, pl.ds(c * 16, 16)]
            rows1_v[i, pl.ds(c * 16, 16)] = ga * r1 + gb * r2
        return carry

    lax.fori_loop(0, TPW, cbody, 0)
    pltpu.sync_copy(rows1_v, out_hbm.at[pl.ds(base_t, TPW)])


def _combine(y, pos1, pos2, g1f, g2f):
    mesh = plsc.VectorSubcoreMesh(core_axis_name="c", subcore_axis_name="s")
    fn = functools.partial(
        pl.kernel,
        out_type=jax.ShapeDtypeStruct((T, DIM), jnp.float32),
        mesh=mesh,
        scratch_types=[
            pltpu.VMEM((TPW,), jnp.int32),
            pltpu.VMEM((TPW,), jnp.int32),
            pltpu.VMEM((TPW,), jnp.float32),
            pltpu.VMEM((TPW,), jnp.float32),
            pltpu.VMEM((TPW, DIM), jnp.float32),
            pltpu.VMEM((TPW, DIM), jnp.float32),
        ],
        compiler_params=pltpu.CompilerParams(needs_layout_passes=False),
    )(_combine_body)
    return fn(y, pos1, pos2, g1f, g2f)


def kernel(x, Wr, br, Wn, bn, W1, b1, W2, b2):
    xf = x.reshape(T, DIM)
    eps = jax.random.normal(jax.random.key(42), (B, S, E),
                            dtype=jnp.float32).reshape(T, E)

    i1, i2, g1, g2 = _router(xf, Wr, br, Wn, bn, eps)
    pos1, pos2, xs, rs, nt = _dispatch(i1.reshape(T), i2.reshape(T), xf)
    y = _ffn(rs, nt, xs, W1, b1, W2, b2)
    out = _combine(y, pos1, pos2, g1.reshape(T), g2.reshape(T))
    return out.reshape(x.shape)


# speedup vs baseline: 1.0664x; 1.0664x over previous
import functools

import jax
import jax.numpy as jnp
from jax import lax
from jax.experimental import pallas as pl
from jax.experimental.pallas import tpu as pltpu
from jax.experimental.pallas import tpu_sc as plsc

DIM = 768
E = 8
B = 1
S = 2048
T = B * S
H = 4 * DIM

BLK = 256
NPAD = 4608
HCH = 2
HB = H // HCH

NW = 32
TPW = T // NW
NEG = -1e30


def _router_body(x_ref, wr_ref, br_ref, wn_ref, bn_ref, eps_ref,
                 i1_ref, i2_ref, g1_ref, g2_ref):
    x = x_ref[...]
    logits = lax.dot_general(
        x, wr_ref[...], (((1,), (1,)), ((), ())),
        preferred_element_type=jnp.float32) + br_ref[...]
    noise = lax.dot_general(
        x, wn_ref[...], (((1,), (1,)), ((), ())),
        preferred_element_type=jnp.float32) + bn_ref[...]
    sp = jnp.log1p(jnp.exp(-jnp.abs(noise))) + jnp.maximum(noise, 0.0)
    noisy = logits + eps_ref[...] * sp

    iota = lax.broadcasted_iota(jnp.int32, (T, E), 1)
    m1 = jnp.max(noisy, axis=1, keepdims=True)
    i1 = jnp.min(jnp.where(noisy == m1, iota, E), axis=1, keepdims=True)
    masked = jnp.where(iota == i1, NEG, noisy)
    m2 = jnp.max(masked, axis=1, keepdims=True)
    i2 = jnp.min(jnp.where(masked == m2, iota, E), axis=1, keepdims=True)
    t = jnp.exp(m2 - m1)
    g1 = 1.0 / (1.0 + t)
    g2 = t / (1.0 + t)
    i1_ref[...] = i1
    i2_ref[...] = i2
    g1_ref[...] = g1
    g2_ref[...] = g2


def _router(xf, Wr, br, Wn, bn, eps):
    return pl.pallas_call(
        _router_body,
        out_shape=[
            jax.ShapeDtypeStruct((T, 1), jnp.int32),
            jax.ShapeDtypeStruct((T, 1), jnp.int32),
            jax.ShapeDtypeStruct((T, 1), jnp.float32),
            jax.ShapeDtypeStruct((T, 1), jnp.float32),
        ],
        in_specs=[
            pl.BlockSpec((T, DIM), lambda: (0, 0)),
            pl.BlockSpec((E, DIM), lambda: (0, 0)),
            pl.BlockSpec((1, E), lambda: (0, 0)),
            pl.BlockSpec((E, DIM), lambda: (0, 0)),
            pl.BlockSpec((1, E), lambda: (0, 0)),
            pl.BlockSpec((T, E), lambda: (0, 0)),
        ],
        out_specs=[
            pl.BlockSpec((T, 1), lambda: (0, 0)),
            pl.BlockSpec((T, 1), lambda: (0, 0)),
            pl.BlockSpec((T, 1), lambda: (0, 0)),
            pl.BlockSpec((T, 1), lambda: (0, 0)),
        ],
    )(xf, Wr, br.reshape(1, E), Wn, bn.reshape(1, E), eps)


def _dispatch_body(i1_hbm, i2_hbm, x_hbm,
                   pos1_hbm, pos2_hbm, xs_hbm, rs_hbm, nt_hbm,
                   ia_v, ib_v, mya_v, myb_v, pos1_v, pos2_v, xrows_v,
                   d0_v, d1_v, semA, semB):
    cid = lax.axis_index("c")
    sid = lax.axis_index("s")
    wid = cid * 16 + sid
    base_t = wid * TPW
    lane = lax.broadcasted_iota(jnp.int32, (16,), 0)
    zero16 = jnp.zeros((16,), jnp.int32)
    one16 = jnp.full((16,), 1, jnp.int32)

    def _splat(s):
        return lax.broadcast_in_dim(s, (16,), ())

    ca = pltpu.async_copy(i1_hbm, ia_v, semA)
    cb = pltpu.async_copy(i2_hbm, ib_v, semA)
    c1 = pltpu.async_copy(i1_hbm.at[pl.ds(base_t, TPW)], mya_v, semB)
    c2 = pltpu.async_copy(i2_hbm.at[pl.ds(base_t, TPW)], myb_v, semB)
    c3 = pltpu.async_copy(x_hbm.at[pl.ds(base_t, TPW)], xrows_v, semB)
    ca.wait()
    cb.wait()

    def hist_body(k, carry):
        va = ia_v[pl.ds(k * 16, 16)]
        vb = ib_v[pl.ds(k * 16, 16)]
        selv = _splat((k < wid * (TPW // 16)).astype(jnp.int32))
        out = []
        for e in range(E):
            inc = (va == e).astype(jnp.int32) + (vb == e).astype(jnp.int32)
            out.append(carry[e] + inc)
            out.append(carry[E + e] + inc * selv)
        return tuple(out[0::2]) + tuple(out[1::2])

    acc = lax.fori_loop(0, T // 16, hist_body, (zero16,) * (2 * E))
    tot = zero16
    pre = zero16
    for e in range(E):
        tot = tot + jnp.where(lane == e, _splat(jnp.sum(acc[e])), zero16)
        pre = pre + jnp.where(lane == e, _splat(jnp.sum(acc[E + e])),
                              zero16)

    nb8 = lax.shift_right_logical(tot + _splat(jnp.int32(7)),
                                  jnp.full((16,), 3, jnp.int32))
    cum8 = plsc.cumsum(nb8)
    seg_base = (cum8 - nb8) * _splat(jnp.int32(8))
    start = seg_base + pre

    @pl.when(wid == 0)
    def _():
        ntile = lax.shift_right_logical(tot + _splat(jnp.int32(BLK - 1)),
                                        jnp.full((16,), 8, jnp.int32))
        d0_v[pl.ds(0, 16)] = seg_base
        d1_v[pl.ds(0, 16)] = ntile
        pltpu.sync_copy(d0_v, rs_hbm)
        pltpu.sync_copy(d1_v, nt_hbm)

    c1.wait()
    c2.wait()
    c3.wait()
    cur = start
    for slot in range(2):
        src = mya_v if slot == 0 else myb_v
        dst = pos1_v if slot == 0 else pos2_v
        for v in range(TPW // 16):
            ev = src[pl.ds(v * 16, 16)]
            pos = zero16
            for e in range(E):
                m = ev == e
                ind = m.astype(jnp.int32)
                r = jnp.cumsum(ind)
                ce = jnp.sum(jnp.where(lane == e, cur, zero16))
                pos = jnp.where(m, _splat(ce) + r - one16, pos)
                cur = cur + jnp.where(lane == e, _splat(jnp.sum(ind)),
                                      zero16)
            dst[pl.ds(v * 16, 16)] = pos

    o1 = pltpu.async_copy(pos1_v, pos1_hbm.at[pl.ds(base_t, TPW)], semB)
    o2 = pltpu.async_copy(pos2_v, pos2_hbm.at[pl.ds(base_t, TPW)], semB)
    o3 = pltpu.async_copy(xrows_v, xs_hbm.at[pos1_v], semB)
    o4 = pltpu.async_copy(xrows_v, xs_hbm.at[pos2_v], semB)
    o1.wait()
    o2.wait()
    o3.wait()
    o4.wait()


def _dispatch(i1f, i2f, xf):
    mesh = plsc.VectorSubcoreMesh(core_axis_name="c", subcore_axis_name="s")
    fn = functools.partial(
        pl.kernel,
        out_type=[
            jax.ShapeDtypeStruct((T,), jnp.int32),
            jax.ShapeDtypeStruct((T,), jnp.int32),
            jax.ShapeDtypeStruct((NPAD, DIM), jnp.float32),
            jax.ShapeDtypeStruct((16,), jnp.int32),
            jax.ShapeDtypeStruct((16,), jnp.int32),
        ],
        mesh=mesh,
        scratch_types=[
            pltpu.VMEM((T,), jnp.int32),
            pltpu.VMEM((T,), jnp.int32),
            pltpu.VMEM((TPW,), jnp.int32),
            pltpu.VMEM((TPW,), jnp.int32),
            pltpu.VMEM((TPW,), jnp.int32),
            pltpu.VMEM((TPW,), jnp.int32),
            pltpu.VMEM((TPW, DIM), jnp.float32),
            pltpu.VMEM((16,), jnp.int32),
            pltpu.VMEM((16,), jnp.int32),
            pltpu.SemaphoreType.DMA,
            pltpu.SemaphoreType.DMA,
        ],
        compiler_params=pltpu.CompilerParams(needs_layout_passes=False),
    )(_dispatch_body)
    return fn(i1f, i2f, xf)


def _ffn_body(rs_ref, nt_ref,
              xs_ref, w1_ref, b1_ref, w2_ref, b2_ref, y_ref):
    e = pl.program_id(0)
    c = pl.program_id(1)
    rs = rs_ref[e]
    nt = nt_ref[e]

    def tile(t, carry):
        s = pl.multiple_of(rs + t * BLK, 8)
        xb = xs_ref[pl.ds(s, BLK), :]
        h = lax.dot_general(
            xb, w1_ref[0], (((1,), (1,)), ((), ())),
            preferred_element_type=jnp.float32) + b1_ref[0]
        h = jnp.maximum(h, 0.0)
        o = lax.dot_general(
            h, w2_ref[0], (((1,), (1,)), ((), ())),
            preferred_element_type=jnp.float32)

        @pl.when(c == 0)
        def _():
            y_ref[pl.ds(s, BLK), :] = o + b2_ref[0]

        @pl.when(c != 0)
        def _():
            y_ref[pl.ds(s, BLK), :] += o

        return carry

    lax.fori_loop(0, nt, tile, 0)


def _ffn(rs, nt, xs, W1, b1, W2, b2):
    grid_spec = pltpu.PrefetchScalarGridSpec(
        num_scalar_prefetch=2,
        grid=(E, HCH),
        in_specs=[
            pl.BlockSpec((NPAD, DIM), lambda e, c, rs, nt: (0, 0)),
            pl.BlockSpec((1, HB, DIM), lambda e, c, rs, nt: (e, c, 0)),
            pl.BlockSpec((1, 1, HB), lambda e, c, rs, nt: (e, 0, c)),
            pl.BlockSpec((1, DIM, HB), lambda e, c, rs, nt: (e, 0, c)),
            pl.BlockSpec((1, 1, DIM), lambda e, c, rs, nt: (e, 0, 0)),
        ],
        out_specs=pl.BlockSpec((NPAD, DIM), lambda e, c, rs, nt: (0, 0)),
    )
    return pl.pallas_call(
        _ffn_body,
        grid_spec=grid_spec,
        out_shape=jax.ShapeDtypeStruct((NPAD, DIM), jnp.float32),
        compiler_params=pltpu.CompilerParams(
            dimension_semantics=("arbitrary", "arbitrary"),
            vmem_limit_bytes=56 * 1024 * 1024,
        ),
    )(rs, nt, xs, W1, b1.reshape(E, 1, H), W2, b2.reshape(E, 1, DIM))


def _combine_body(y_hbm, pos1_hbm, pos2_hbm, g1_hbm, g2_hbm, out_hbm,
                  idx1_v, idx2_v, g1_v, g2_v, rows1_v, rows2_v, semA, semB):
    cid = lax.axis_index("c")
    sid = lax.axis_index("s")
    wid = cid * 16 + sid
    base_t = wid * TPW

    c1 = pltpu.async_copy(pos1_hbm.at[pl.ds(base_t, TPW)], idx1_v, semA)
    c2 = pltpu.async_copy(pos2_hbm.at[pl.ds(base_t, TPW)], idx2_v, semA)
    c3 = pltpu.async_copy(g1_hbm.at[pl.ds(base_t, TPW)], g1_v, semA)
    c4 = pltpu.async_copy(g2_hbm.at[pl.ds(base_t, TPW)], g2_v, semA)
    c1.wait()
    c2.wait()
    g1c = pltpu.async_copy(y_hbm.at[idx1_v], rows1_v, semB)
    g2c = pltpu.async_copy(y_hbm.at[idx2_v], rows2_v, semB)
    c3.wait()
    c4.wait()
    g1c.wait()
    g2c.wait()

    def cbody(i, carry):
        isplat = jnp.broadcast_to(i, (16,))
        ga = plsc.load_gather(g1_v, [isplat])
        gb = plsc.load_gather(g2_v, [isplat])
        for c in range(DIM // 16):
            r1 = rows1_v[i, pl.ds(c * 16, 16)]
            r2 = rows2_v[i, pl.ds(c * 16, 16)]
            rows1_v[i, pl.ds(c * 16, 16)] = ga * r1 + gb * r2
        return carry

    lax.fori_loop(0, TPW, cbody, 0)
    pltpu.sync_copy(rows1_v, out_hbm.at[pl.ds(base_t, TPW)])


def _combine(y, pos1, pos2, g1f, g2f):
    mesh = plsc.VectorSubcoreMesh(core_axis_name="c", subcore_axis_name="s")
    fn = functools.partial(
        pl.kernel,
        out_type=jax.ShapeDtypeStruct((T, DIM), jnp.float32),
        mesh=mesh,
        scratch_types=[
            pltpu.VMEM((TPW,), jnp.int32),
            pltpu.VMEM((TPW,), jnp.int32),
            pltpu.VMEM((TPW,), jnp.float32),
            pltpu.VMEM((TPW,), jnp.float32),
            pltpu.VMEM((TPW, DIM), jnp.float32),
            pltpu.VMEM((TPW, DIM), jnp.float32),
            pltpu.SemaphoreType.DMA,
            pltpu.SemaphoreType.DMA,
        ],
        compiler_params=pltpu.CompilerParams(needs_layout_passes=False),
    )(_combine_body)
    return fn(y, pos1, pos2, g1f, g2f)


def kernel(x, Wr, br, Wn, bn, W1, b1, W2, b2):
    xf = x.reshape(T, DIM)
    eps = jax.random.normal(jax.random.key(42), (B, S, E),
                            dtype=jnp.float32).reshape(T, E)

    i1, i2, g1, g2 = _router(xf, Wr, br, Wn, bn, eps)
    pos1, pos2, xs, rs, nt = _dispatch(i1.reshape(T), i2.reshape(T), xf)
    y = _ffn(rs, nt, xs, W1, b1, W2, b2)
    out = _combine(y, pos1, pos2, g1.reshape(T), g2.reshape(T))
    return out.reshape(x.shape)


# speedup vs baseline: 1.0865x; 1.0189x over previous
import functools

import jax
import jax.numpy as jnp
from jax import lax
from jax.experimental import pallas as pl
from jax.experimental.pallas import tpu as pltpu
from jax.experimental.pallas import tpu_sc as plsc

DIM = 768
E = 8
B = 1
S = 2048
T = B * S
H = 4 * DIM

BLK = 256
NPAD = 4608
HCH = 2
HB = H // HCH

NW = 32
TPW = T // NW
NEG = -1e30


def _router_body(x_ref, wr_ref, br_ref, wn_ref, bn_ref, eps_ref,
                 i1_ref, i2_ref, g1_ref, g2_ref):
    x = x_ref[...]
    logits = lax.dot_general(
        x, wr_ref[...], (((1,), (1,)), ((), ())),
        preferred_element_type=jnp.float32) + br_ref[...]
    noise = lax.dot_general(
        x, wn_ref[...], (((1,), (1,)), ((), ())),
        preferred_element_type=jnp.float32) + bn_ref[...]
    sp = jnp.log1p(jnp.exp(-jnp.abs(noise))) + jnp.maximum(noise, 0.0)
    noisy = logits + eps_ref[...] * sp

    iota = lax.broadcasted_iota(jnp.int32, (T, E), 1)
    m1 = jnp.max(noisy, axis=1, keepdims=True)
    i1 = jnp.min(jnp.where(noisy == m1, iota, E), axis=1, keepdims=True)
    masked = jnp.where(iota == i1, NEG, noisy)
    m2 = jnp.max(masked, axis=1, keepdims=True)
    i2 = jnp.min(jnp.where(masked == m2, iota, E), axis=1, keepdims=True)
    t = jnp.exp(m2 - m1)
    g1 = 1.0 / (1.0 + t)
    g2 = t / (1.0 + t)
    i1_ref[...] = i1
    i2_ref[...] = i2
    g1_ref[...] = g1
    g2_ref[...] = g2


def _router(xf, Wr, br, Wn, bn, eps):
    return pl.pallas_call(
        _router_body,
        out_shape=[
            jax.ShapeDtypeStruct((T, 1), jnp.int32),
            jax.ShapeDtypeStruct((T, 1), jnp.int32),
            jax.ShapeDtypeStruct((T, 1), jnp.float32),
            jax.ShapeDtypeStruct((T, 1), jnp.float32),
        ],
        in_specs=[
            pl.BlockSpec((T, DIM), lambda: (0, 0)),
            pl.BlockSpec((E, DIM), lambda: (0, 0)),
            pl.BlockSpec((1, E), lambda: (0, 0)),
            pl.BlockSpec((E, DIM), lambda: (0, 0)),
            pl.BlockSpec((1, E), lambda: (0, 0)),
            pl.BlockSpec((T, E), lambda: (0, 0)),
        ],
        out_specs=[
            pl.BlockSpec((T, 1), lambda: (0, 0)),
            pl.BlockSpec((T, 1), lambda: (0, 0)),
            pl.BlockSpec((T, 1), lambda: (0, 0)),
            pl.BlockSpec((T, 1), lambda: (0, 0)),
        ],
    )(xf, Wr, br.reshape(1, E), Wn, bn.reshape(1, E), eps)


def _dispatch_body(i1_hbm, i2_hbm, x_hbm,
                   pos1_hbm, pos2_hbm, xs_hbm, rs_hbm, nt_hbm,
                   ia_v, ib_v, mya_v, myb_v, pos1_v, pos2_v, xrows_v,
                   d0_v, d1_v, semA, semB):
    cid = lax.axis_index("c")
    sid = lax.axis_index("s")
    wid = cid * 16 + sid
    base_t = wid * TPW
    lane = lax.broadcasted_iota(jnp.int32, (16,), 0)
    zero16 = jnp.zeros((16,), jnp.int32)
    one16 = jnp.full((16,), 1, jnp.int32)

    def _splat(s):
        return lax.broadcast_in_dim(s, (16,), ())

    ca = pltpu.async_copy(i1_hbm, ia_v, semA)
    cb = pltpu.async_copy(i2_hbm, ib_v, semA)
    c1 = pltpu.async_copy(i1_hbm.at[pl.ds(base_t, TPW)], mya_v, semB)
    c2 = pltpu.async_copy(i2_hbm.at[pl.ds(base_t, TPW)], myb_v, semB)
    c3 = pltpu.async_copy(x_hbm.at[pl.ds(base_t, TPW)], xrows_v, semB)
    ca.wait()
    cb.wait()

    def hist_body(k, carry):
        va = ia_v[pl.ds(k * 16, 16)]
        vb = ib_v[pl.ds(k * 16, 16)]
        selv = _splat((k < wid * (TPW // 16)).astype(jnp.int32))
        out = []
        for e in range(E):
            inc = (va == e).astype(jnp.int32) + (vb == e).astype(jnp.int32)
            out.append(carry[e] + inc)
            out.append(carry[E + e] + inc * selv)
        return tuple(out[0::2]) + tuple(out[1::2])

    acc = lax.fori_loop(0, T // 16, hist_body, (zero16,) * (2 * E))
    tot = zero16
    pre = zero16
    for e in range(E):
        tot = tot + jnp.where(lane == e, _splat(jnp.sum(acc[e])), zero16)
        pre = pre + jnp.where(lane == e, _splat(jnp.sum(acc[E + e])),
                              zero16)

    nb8 = lax.shift_right_logical(tot + _splat(jnp.int32(7)),
                                  jnp.full((16,), 3, jnp.int32))
    cum8 = plsc.cumsum(nb8)
    seg_base = (cum8 - nb8) * _splat(jnp.int32(8))
    start = seg_base + pre

    @pl.when(wid == 0)
    def _():
        ntile = lax.shift_right_logical(tot + _splat(jnp.int32(BLK - 1)),
                                        jnp.full((16,), 8, jnp.int32))
        d0_v[pl.ds(0, 16)] = seg_base
        d1_v[pl.ds(0, 16)] = ntile
        pltpu.sync_copy(d0_v, rs_hbm)
        pltpu.sync_copy(d1_v, nt_hbm)

    c1.wait()
    c2.wait()
    c3.wait()
    cur = start
    outs = []
    for slot in range(2):
        src = mya_v if slot == 0 else myb_v
        dst = pos1_v if slot == 0 else pos2_v
        for v in range(TPW // 16):
            ev = src[pl.ds(v * 16, 16)]
            pos = zero16
            for e in range(E):
                m = ev == e
                ind = m.astype(jnp.int32)
                r = jnp.cumsum(ind)
                ce = jnp.sum(jnp.where(lane == e, cur, zero16))
                pos = jnp.where(m, _splat(ce) + r - one16, pos)
                cur = cur + jnp.where(lane == e, _splat(jnp.sum(ind)),
                                      zero16)
            dst[pl.ds(v * 16, 16)] = pos
        pos_hbm = pos1_hbm if slot == 0 else pos2_hbm
        outs.append(pltpu.async_copy(dst, pos_hbm.at[pl.ds(base_t, TPW)],
                                     semB))
        outs.append(pltpu.async_copy(xrows_v, xs_hbm.at[dst], semB))
    for o in outs:
        o.wait()


def _dispatch(i1f, i2f, xf):
    mesh = plsc.VectorSubcoreMesh(core_axis_name="c", subcore_axis_name="s")
    fn = functools.partial(
        pl.kernel,
        out_type=[
            jax.ShapeDtypeStruct((T,), jnp.int32),
            jax.ShapeDtypeStruct((T,), jnp.int32),
            jax.ShapeDtypeStruct((NPAD, DIM), jnp.float32),
            jax.ShapeDtypeStruct((16,), jnp.int32),
            jax.ShapeDtypeStruct((16,), jnp.int32),
        ],
        mesh=mesh,
        scratch_types=[
            pltpu.VMEM((T,), jnp.int32),
            pltpu.VMEM((T,), jnp.int32),
            pltpu.VMEM((TPW,), jnp.int32),
            pltpu.VMEM((TPW,), jnp.int32),
            pltpu.VMEM((TPW,), jnp.int32),
            pltpu.VMEM((TPW,), jnp.int32),
            pltpu.VMEM((TPW, DIM), jnp.float32),
            pltpu.VMEM((16,), jnp.int32),
            pltpu.VMEM((16,), jnp.int32),
            pltpu.SemaphoreType.DMA,
            pltpu.SemaphoreType.DMA,
        ],
        compiler_params=pltpu.CompilerParams(needs_layout_passes=False),
    )(_dispatch_body)
    return fn(i1f, i2f, xf)


def _ffn_body(rs_ref, nt_ref,
              xs_ref, w1_ref, b1_ref, w2_ref, b2_ref, y_ref):
    e = pl.program_id(0)
    c = pl.program_id(1)
    rs = rs_ref[e]
    nt = nt_ref[e]

    def tile(t, carry):
        s = pl.multiple_of(rs + t * BLK, 8)
        xb = xs_ref[pl.ds(s, BLK), :]
        h = lax.dot_general(
            xb, w1_ref[0], (((1,), (1,)), ((), ())),
            preferred_element_type=jnp.float32) + b1_ref[0]
        h = jnp.maximum(h, 0.0)
        o = lax.dot_general(
            h, w2_ref[0], (((1,), (1,)), ((), ())),
            preferred_element_type=jnp.float32)

        @pl.when(c == 0)
        def _():
            y_ref[pl.ds(s, BLK), :] = o + b2_ref[0]

        @pl.when(c != 0)
        def _():
            y_ref[pl.ds(s, BLK), :] += o

        return carry

    lax.fori_loop(0, nt, tile, 0)


def _ffn(rs, nt, xs, W1, b1, W2, b2):
    grid_spec = pltpu.PrefetchScalarGridSpec(
        num_scalar_prefetch=2,
        grid=(E, HCH),
        in_specs=[
            pl.BlockSpec((NPAD, DIM), lambda e, c, rs, nt: (0, 0)),
            pl.BlockSpec((1, HB, DIM), lambda e, c, rs, nt: (e, c, 0)),
            pl.BlockSpec((1, 1, HB), lambda e, c, rs, nt: (e, 0, c)),
            pl.BlockSpec((1, DIM, HB), lambda e, c, rs, nt: (e, 0, c)),
            pl.BlockSpec((1, 1, DIM), lambda e, c, rs, nt: (e, 0, 0)),
        ],
        out_specs=pl.BlockSpec((NPAD, DIM), lambda e, c, rs, nt: (0, 0)),
    )
    return pl.pallas_call(
        _ffn_body,
        grid_spec=grid_spec,
        out_shape=jax.ShapeDtypeStruct((NPAD, DIM), jnp.float32),
        compiler_params=pltpu.CompilerParams(
            dimension_semantics=("arbitrary", "arbitrary"),
            vmem_limit_bytes=56 * 1024 * 1024,
        ),
    )(rs, nt, xs, W1, b1.reshape(E, 1, H), W2, b2.reshape(E, 1, DIM))


def _combine_body(y_hbm, pos1_hbm, pos2_hbm, g1_hbm, g2_hbm, out_hbm,
                  idx1_v, idx2_v, g1_v, g2_v, rows1_v, rows2_v, semA, semB):
    cid = lax.axis_index("c")
    sid = lax.axis_index("s")
    wid = cid * 16 + sid
    base_t = wid * TPW

    c1 = pltpu.async_copy(pos1_hbm.at[pl.ds(base_t, TPW)], idx1_v, semA)
    c2 = pltpu.async_copy(pos2_hbm.at[pl.ds(base_t, TPW)], idx2_v, semA)
    c3 = pltpu.async_copy(g1_hbm.at[pl.ds(base_t, TPW)], g1_v, semA)
    c4 = pltpu.async_copy(g2_hbm.at[pl.ds(base_t, TPW)], g2_v, semA)
    c1.wait()
    c2.wait()
    HALF = TPW // 2
    h1a = pltpu.async_copy(y_hbm.at[idx1_v.at[pl.ds(0, HALF)]],
                           rows1_v.at[pl.ds(0, HALF)], semB)
    h1b = pltpu.async_copy(y_hbm.at[idx2_v.at[pl.ds(0, HALF)]],
                           rows2_v.at[pl.ds(0, HALF)], semB)
    h2a = pltpu.async_copy(y_hbm.at[idx1_v.at[pl.ds(HALF, HALF)]],
                           rows1_v.at[pl.ds(HALF, HALF)], semB)
    h2b = pltpu.async_copy(y_hbm.at[idx2_v.at[pl.ds(HALF, HALF)]],
                           rows2_v.at[pl.ds(HALF, HALF)], semB)
    c3.wait()
    c4.wait()

    def cbody(i, carry):
        isplat = jnp.broadcast_to(i, (16,))
        ga = plsc.load_gather(g1_v, [isplat])
        gb = plsc.load_gather(g2_v, [isplat])
        for c in range(DIM // 16):
            r1 = rows1_v[i, pl.ds(c * 16, 16)]
            r2 = rows2_v[i, pl.ds(c * 16, 16)]
            rows1_v[i, pl.ds(c * 16, 16)] = ga * r1 + gb * r2
        return carry

    h1a.wait()
    h1b.wait()
    lax.fori_loop(0, HALF, cbody, 0)
    o1 = pltpu.async_copy(rows1_v.at[pl.ds(0, HALF)],
                          out_hbm.at[pl.ds(base_t, HALF)], semA)
    h2a.wait()
    h2b.wait()
    lax.fori_loop(HALF, TPW, cbody, 0)
    o1.wait()
    pltpu.sync_copy(rows1_v.at[pl.ds(HALF, HALF)],
                    out_hbm.at[pl.ds(base_t + HALF, HALF)])


def _combine(y, pos1, pos2, g1f, g2f):
    mesh = plsc.VectorSubcoreMesh(core_axis_name="c", subcore_axis_name="s")
    fn = functools.partial(
        pl.kernel,
        out_type=jax.ShapeDtypeStruct((T, DIM), jnp.float32),
        mesh=mesh,
        scratch_types=[
            pltpu.VMEM((TPW,), jnp.int32),
            pltpu.VMEM((TPW,), jnp.int32),
            pltpu.VMEM((TPW,), jnp.float32),
            pltpu.VMEM((TPW,), jnp.float32),
            pltpu.VMEM((TPW, DIM), jnp.float32),
            pltpu.VMEM((TPW, DIM), jnp.float32),
            pltpu.SemaphoreType.DMA,
            pltpu.SemaphoreType.DMA,
        ],
        compiler_params=pltpu.CompilerParams(needs_layout_passes=False),
    )(_combine_body)
    return fn(y, pos1, pos2, g1f, g2f)


def kernel(x, Wr, br, Wn, bn, W1, b1, W2, b2):
    xf = x.reshape(T, DIM)
    eps = jax.random.normal(jax.random.key(42), (B, S, E),
                            dtype=jnp.float32).reshape(T, E)

    i1, i2, g1, g2 = _router(xf, Wr, br, Wn, bn, eps)
    pos1, pos2, xs, rs, nt = _dispatch(i1.reshape(T), i2.reshape(T), xf)
    y = _ffn(rs, nt, xs, W1, b1, W2, b2)
    out = _combine(y, pos1, pos2, g1.reshape(T), g2.reshape(T))
    return out.reshape(x.shape)
